# baseline (speedup 1.0000x reference)
import functools

import jax
import jax.numpy as jnp
from jax import lax
from jax.experimental import pallas as pl
from jax.experimental.pallas import tpu as pltpu
from jax.experimental.pallas import tpu_sc as plsc

_D = 256
_K = 8192
_N = 16384
_BETA = 0.25

_NT = 512
_KT = 256
_CHUNK_STARTS = (0, 11, 22)
_NKS = _K // _KT


def _bf16_round(v):
    return v.astype(jnp.bfloat16).astype(jnp.float32)


def _argmin_body(x2_ref, flat_ref, emb_ref, e2_ref, codes_ref,
                 min_s, idx_s, gmin_s, gidx_s):
    k = pl.program_id(1)
    mm = jax.lax.dot_general(
        flat_ref[...], emb_ref[...], (((1,), (0,)), ((), ())),
        preferred_element_type=jnp.float32)
    d = (x2_ref[...] - 2.0 * mm) + e2_ref[...]
    lmin = jnp.min(d, axis=1, keepdims=True)
    iota = jax.lax.broadcasted_iota(jnp.int32, d.shape, 1)
    lidx = jnp.min(jnp.where(d == lmin, iota, _K), axis=1, keepdims=True) + k * _KT

    is_start = (k == _CHUNK_STARTS[0]) | (k == _CHUNK_STARTS[1]) | (k == _CHUNK_STARTS[2])

    @pl.when(is_start)
    def _():
        min_s[...] = lmin
        idx_s[...] = lidx

    @pl.when(~is_start)
    def _():
        upd = lmin < min_s[...]
        min_s[...] = jnp.where(upd, lmin, min_s[...])
        idx_s[...] = jnp.where(upd, lidx, idx_s[...])

    @pl.when(k == _CHUNK_STARTS[1] - 1)
    def _():
        gmin_s[...] = _bf16_round(min_s[...])
        gidx_s[...] = idx_s[...]

    @pl.when((k == _CHUNK_STARTS[2] - 1) | (k == _NKS - 1))
    def _():
        upd = min_s[...] < gmin_s[...]
        gmin_s[...] = jnp.where(upd, _bf16_round(min_s[...]), gmin_s[...])
        gidx_s[...] = jnp.where(upd, idx_s[...], gidx_s[...])

    @pl.when(k == _NKS - 1)
    def _():
        codes_ref[...] = gidx_s[...]


def _compute_codes(flat, embeddings, x2, e2):
    grid = (_N // _NT, _NKS)
    codes = pl.pallas_call(
        _argmin_body,
        grid=grid,
        in_specs=[
            pl.BlockSpec((_NT, 1), lambda i, k: (i, 0)),
            pl.BlockSpec((_NT, _D), lambda i, k: (i, 0)),
            pl.BlockSpec((_D, _KT), lambda i, k: (0, k)),
            pl.BlockSpec((1, _KT), lambda i, k: (0, k)),
        ],
        out_specs=pl.BlockSpec((_NT, 1), lambda i, k: (i, 0)),
        out_shape=jax.ShapeDtypeStruct((_N, 1), jnp.int32),
        scratch_shapes=[
            pltpu.VMEM((_NT, 1), jnp.float32),
            pltpu.VMEM((_NT, 1), jnp.int32),
            pltpu.VMEM((_NT, 1), jnp.float32),
            pltpu.VMEM((_NT, 1), jnp.int32),
        ],
    )(x2, flat, embeddings, e2)
    return codes[:, 0]


_SC_INFO = plsc.get_sparse_core_info()
_NC = _SC_INFO.num_cores
_NS = _SC_INFO.num_subcores
_NW = _NC * _NS
_BPW = _N // _NW
_GCH = 128


@functools.partial(
    pl.kernel,
    out_type=jax.ShapeDtypeStruct((_N, _D), jnp.float32),
    scratch_types=[
        pltpu.VMEM((_GCH,), jnp.int32),
        pltpu.VMEM((_GCH, _D), jnp.float32),
        pltpu.SemaphoreType.DMA,
    ],
    mesh=plsc.VectorSubcoreMesh(core_axis_name="c", subcore_axis_name="s"),
)
def _gather_rows(codes_hbm, table_hbm, out_hbm, idx_v, rows_v, sem):
    wid = lax.axis_index("s") * _NC + lax.axis_index("c")
    base = wid * _BPW
    for c in range(_BPW // _GCH):
        off = base + c * _GCH
        pltpu.sync_copy(codes_hbm.at[pl.ds(off, _GCH)], idx_v)
        pltpu.async_copy(table_hbm.at[idx_v], rows_v, sem).wait()
        pltpu.sync_copy(rows_v, out_hbm.at[pl.ds(off, _GCH)])


def _out_loss_body(x_ref, q_ref, out_ref, loss_ref, acc_s):
    i = pl.program_id(0)
    xv = x_ref[...]
    qv = q_ref[...]
    out_ref[...] = xv + (qv - xv)
    diff = xv - qv
    part = jnp.sum(diff * diff)

    @pl.when(i == 0)
    def _():
        acc_s[0, 0] = part

    @pl.when(i > 0)
    def _():
        acc_s[0, 0] = acc_s[0, 0] + part

    @pl.when(i == pl.num_programs(0) - 1)
    def _():
        total = acc_s[0, 0]
        mean = total / jnp.float32(_N * _D)
        loss_ref[...] = jnp.reshape(mean + jnp.float32(_BETA) * mean, (1, 1))


def _out_and_loss(flat_x, q):
    out, loss = pl.pallas_call(
        _out_loss_body,
        grid=(_N // _NT,),
        in_specs=[
            pl.BlockSpec((_NT, _D), lambda i: (i, 0)),
            pl.BlockSpec((_NT, _D), lambda i: (i, 0)),
        ],
        out_specs=[
            pl.BlockSpec((_NT, _D), lambda i: (i, 0)),
            pl.BlockSpec((1, 1), lambda i: (0, 0)),
        ],
        out_shape=[
            jax.ShapeDtypeStruct((_N, _D), jnp.float32),
            jax.ShapeDtypeStruct((1, 1), jnp.float32),
        ],
        scratch_shapes=[pltpu.SMEM((1, 1), jnp.float32)],
    )(flat_x, q)
    return out, loss[0, 0]


def kernel(x, embeddings):
    flat = jnp.reshape(x, (-1, _D))
    x2 = jnp.sum(flat ** 2, axis=1, keepdims=True)
    e2 = jnp.sum(embeddings ** 2, axis=0, keepdims=True)
    codes = _compute_codes(flat, embeddings, x2, e2)
    table = embeddings.T.astype(jnp.bfloat16).astype(jnp.float32)
    q = _gather_rows(codes, table)
    out_flat, loss = _out_and_loss(flat, q)
    return jnp.reshape(out_flat, x.shape), loss


# speedup vs baseline: 2.3130x; 2.3130x over previous
import functools

import jax
import jax.numpy as jnp
from jax import lax
from jax.experimental import pallas as pl
from jax.experimental.pallas import tpu as pltpu
from jax.experimental.pallas import tpu_sc as plsc

_D = 256
_K = 8192
_N = 16384
_BETA = 0.25

_NT = 512
_KT = 1024
_NKS = _K // _KT
_B1_STEP, _B1_ROW = 2, 768
_B2_STEP, _B2_ROW = 5, 512
_BIG = 2 ** 30


def _bf16_round(v):
    return v.astype(jnp.bfloat16).astype(jnp.float32)


def _colmin(d, base):
    lmin = jnp.min(d, axis=0, keepdims=True)
    iota = lax.broadcasted_iota(jnp.int32, d.shape, 0)
    lidx = jnp.min(jnp.where(d == lmin, iota + base, _BIG), axis=0, keepdims=True)
    return lmin, lidx


def _argmin_body(x2_ref, flatT_ref, embT_ref, e2_ref, codes_ref,
                 min_s, idx_s, gmin_s, gidx_s):
    k = pl.program_id(1)
    mm = jax.lax.dot_general(
        embT_ref[...], flatT_ref[...], (((1,), (0,)), ((), ())),
        preferred_element_type=jnp.float32)
    d = (x2_ref[...] + mm) + e2_ref[...]
    base = k * _KT

    def merge_chunk(lmin, lidx):
        upd = lmin < min_s[...]
        min_s[...] = jnp.where(upd, lmin, min_s[...])
        idx_s[...] = jnp.where(upd, lidx, idx_s[...])

    def fold_global(first):
        if first:
            gmin_s[...] = _bf16_round(min_s[...])
            gidx_s[...] = idx_s[...]
        else:
            upd = min_s[...] < gmin_s[...]
            gmin_s[...] = jnp.where(upd, _bf16_round(min_s[...]), gmin_s[...])
            gidx_s[...] = jnp.where(upd, idx_s[...], gidx_s[...])

    @pl.when(k == 0)
    def _():
        lmin, lidx = _colmin(d, base)
        min_s[...] = lmin
        idx_s[...] = lidx

    @pl.when((k == 1) | (k == 3) | (k == 4) | (k == 6) | (k == _NKS - 1))
    def _():
        merge_chunk(*_colmin(d, base))

    @pl.when(k == _B1_STEP)
    def _():
        merge_chunk(*_colmin(d[:_B1_ROW], base))
        fold_global(True)
        lmin, lidx = _colmin(d[_B1_ROW:], base + _B1_ROW)
        min_s[...] = lmin
        idx_s[...] = lidx

    @pl.when(k == _B2_STEP)
    def _():
        merge_chunk(*_colmin(d[:_B2_ROW], base))
        fold_global(False)
        lmin, lidx = _colmin(d[_B2_ROW:], base + _B2_ROW)
        min_s[...] = lmin
        idx_s[...] = lidx

    @pl.when(k == _NKS - 1)
    def _():
        fold_global(False)
        codes_ref[...] = gidx_s[...]


def _compute_codes(flatm2T_bf, embT_bf, x2r, e2c):
    grid = (_N // _NT, _NKS)
    codes = pl.pallas_call(
        _argmin_body,
        grid=grid,
        in_specs=[
            pl.BlockSpec((1, _NT), lambda i, k: (0, i)),
            pl.BlockSpec((_D, _NT), lambda i, k: (0, i)),
            pl.BlockSpec((_KT, _D), lambda i, k: (k, 0)),
            pl.BlockSpec((_KT, 1), lambda i, k: (k, 0)),
        ],
        out_specs=pl.BlockSpec((1, _NT), lambda i, k: (0, i)),
        out_shape=jax.ShapeDtypeStruct((1, _N), jnp.int32),
        scratch_shapes=[
            pltpu.VMEM((1, _NT), jnp.float32),
            pltpu.VMEM((1, _NT), jnp.int32),
            pltpu.VMEM((1, _NT), jnp.float32),
            pltpu.VMEM((1, _NT), jnp.int32),
        ],
    )(x2r, flatm2T_bf, embT_bf, e2c)
    return jnp.reshape(codes, (_N,))


_SC_INFO = plsc.get_sparse_core_info()
_NC = _SC_INFO.num_cores
_NS = _SC_INFO.num_subcores
_NW = _NC * _NS
_BPW = _N // _NW
_GCH = 128


@functools.partial(
    pl.kernel,
    out_type=jax.ShapeDtypeStruct((_N, _D), jnp.float32),
    scratch_types=[
        pltpu.VMEM((_GCH,), jnp.int32),
        pltpu.VMEM((_GCH, _D), jnp.float32),
        pltpu.SemaphoreType.DMA,
    ],
    mesh=plsc.VectorSubcoreMesh(core_axis_name="c", subcore_axis_name="s"),
)
def _gather_rows(codes_hbm, table_hbm, out_hbm, idx_v, rows_v, sem):
    wid = lax.axis_index("s") * _NC + lax.axis_index("c")
    base = wid * _BPW
    for c in range(_BPW // _GCH):
        off = base + c * _GCH
        pltpu.sync_copy(codes_hbm.at[pl.ds(off, _GCH)], idx_v)
        pltpu.async_copy(table_hbm.at[idx_v], rows_v, sem).wait()
        pltpu.sync_copy(rows_v, out_hbm.at[pl.ds(off, _GCH)])


def _out_loss_body(x_ref, q_ref, out_ref, loss_ref, acc_s):
    i = pl.program_id(0)
    xv = x_ref[...]
    qv = q_ref[...]
    out_ref[...] = xv + (qv - xv)
    diff = xv - qv
    part = jnp.sum(diff * diff)

    @pl.when(i == 0)
    def _():
        acc_s[0, 0] = part

    @pl.when(i > 0)
    def _():
        acc_s[0, 0] = acc_s[0, 0] + part

    @pl.when(i == pl.num_programs(0) - 1)
    def _():
        total = acc_s[0, 0]
        mean = total / jnp.float32(_N * _D)
        loss_ref[...] = jnp.reshape(mean + jnp.float32(_BETA) * mean, (1, 1))


def _out_and_loss(flat_x, q):
    out, loss = pl.pallas_call(
        _out_loss_body,
        grid=(_N // _NT,),
        in_specs=[
            pl.BlockSpec((_NT, _D), lambda i: (i, 0)),
            pl.BlockSpec((_NT, _D), lambda i: (i, 0)),
        ],
        out_specs=[
            pl.BlockSpec((_NT, _D), lambda i: (i, 0)),
            pl.BlockSpec((1, 1), lambda i: (0, 0)),
        ],
        out_shape=[
            jax.ShapeDtypeStruct((_N, _D), jnp.float32),
            jax.ShapeDtypeStruct((1, 1), jnp.float32),
        ],
        scratch_shapes=[pltpu.SMEM((1, 1), jnp.float32)],
    )(flat_x, q)
    return out, loss[0, 0]


def kernel(x, embeddings):
    flat = jnp.reshape(x, (-1, _D))
    x2 = jnp.sum(flat ** 2, axis=1, keepdims=True)
    e2 = jnp.sum(embeddings ** 2, axis=0, keepdims=True)
    emb_bf = embeddings.astype(jnp.bfloat16)
    flatm2T_bf = (-2.0 * flat.T).astype(jnp.bfloat16)
    embT_bf = emb_bf.T
    x2r = jnp.reshape(x2, (1, _N))
    e2c = jnp.reshape(e2, (_K, 1))
    codes = _compute_codes(flatm2T_bf, embT_bf, x2r, e2c)
    table = embT_bf.astype(jnp.float32)
    q = _gather_rows(codes, table)
    out_flat, loss = _out_and_loss(flat, q)
    return jnp.reshape(out_flat, x.shape), loss


# speedup vs baseline: 2.3784x; 1.0283x over previous
import functools

import jax
import jax.numpy as jnp
from jax import lax
from jax.experimental import pallas as pl
from jax.experimental.pallas import tpu as pltpu
from jax.experimental.pallas import tpu_sc as plsc

_D = 256
_K = 8192
_N = 16384
_BETA = 0.25

_NT = 512
_KT = 1024
_NKS = _K // _KT
_B1_STEP, _B1_ROW = 2, 768
_B2_STEP, _B2_ROW = 5, 512
_BIG = 3.0e38


def _bf16_round(v):
    return v.astype(jnp.bfloat16).astype(jnp.float32)


def _colmin(d, base):
    lmin = jnp.min(d, axis=0, keepdims=True)
    iota = lax.broadcasted_iota(jnp.int32, (d.shape[0], 1), 0).astype(jnp.float32)
    lidx = jnp.min(jnp.where(d == lmin, iota, _BIG), axis=0, keepdims=True)
    return lmin, lidx + jnp.float32(base)


def _argmin_body(x2_ref, flatT_ref, embT_ref, e2_ref, codes_ref,
                 min_s, idx_s, gmin_s, gidx_s):
    k = pl.program_id(1)
    mm = jax.lax.dot_general(
        embT_ref[...], flatT_ref[...], (((1,), (0,)), ((), ())),
        preferred_element_type=jnp.float32)
    d = (x2_ref[...] + mm) + e2_ref[...]
    base = k * _KT

    def merge_chunk(lmin, lidx):
        upd = lmin < min_s[...]
        min_s[...] = jnp.where(upd, lmin, min_s[...])
        idx_s[...] = jnp.where(upd, lidx, idx_s[...])

    def fold_global(first):
        if first:
            gmin_s[...] = _bf16_round(min_s[...])
            gidx_s[...] = idx_s[...]
        else:
            upd = min_s[...] < gmin_s[...]
            gmin_s[...] = jnp.where(upd, _bf16_round(min_s[...]), gmin_s[...])
            gidx_s[...] = jnp.where(upd, idx_s[...], gidx_s[...])

    @pl.when(k == 0)
    def _():
        lmin, lidx = _colmin(d, base)
        min_s[...] = lmin
        idx_s[...] = lidx

    @pl.when((k == 1) | (k == 3) | (k == 4) | (k == 6) | (k == _NKS - 1))
    def _():
        merge_chunk(*_colmin(d, base))

    @pl.when(k == _B1_STEP)
    def _():
        merge_chunk(*_colmin(d[:_B1_ROW], base))
        fold_global(True)
        lmin, lidx = _colmin(d[_B1_ROW:], base + _B1_ROW)
        min_s[...] = lmin
        idx_s[...] = lidx

    @pl.when(k == _B2_STEP)
    def _():
        merge_chunk(*_colmin(d[:_B2_ROW], base))
        fold_global(False)
        lmin, lidx = _colmin(d[_B2_ROW:], base + _B2_ROW)
        min_s[...] = lmin
        idx_s[...] = lidx

    @pl.when(k == _NKS - 1)
    def _():
        fold_global(False)
        codes_ref[...] = gidx_s[...].astype(jnp.int32)


def _compute_codes(flatm2T_bf, embT_bf, x2r, e2c):
    grid = (_N // _NT, _NKS)
    codes = pl.pallas_call(
        _argmin_body,
        grid=grid,
        in_specs=[
            pl.BlockSpec((1, _NT), lambda i, k: (0, i)),
            pl.BlockSpec((_D, _NT), lambda i, k: (0, i)),
            pl.BlockSpec((_KT, _D), lambda i, k: (k, 0)),
            pl.BlockSpec((_KT, 1), lambda i, k: (k, 0)),
        ],
        out_specs=pl.BlockSpec((1, _NT), lambda i, k: (0, i)),
        out_shape=jax.ShapeDtypeStruct((1, _N), jnp.int32),
        scratch_shapes=[
            pltpu.VMEM((1, _NT), jnp.float32),
            pltpu.VMEM((1, _NT), jnp.float32),
            pltpu.VMEM((1, _NT), jnp.float32),
            pltpu.VMEM((1, _NT), jnp.float32),
        ],
    )(x2r, flatm2T_bf, embT_bf, e2c)
    return jnp.reshape(codes, (_N,))


_SC_INFO = plsc.get_sparse_core_info()
_NC = _SC_INFO.num_cores
_NS = _SC_INFO.num_subcores
_NW = _NC * _NS
_BPW = _N // _NW
_GCH = 128


@functools.partial(
    pl.kernel,
    out_type=jax.ShapeDtypeStruct((_N, _D), jnp.float32),
    scratch_types=[
        pltpu.VMEM((_GCH,), jnp.int32),
        pltpu.VMEM((_GCH, _D), jnp.float32),
        pltpu.SemaphoreType.DMA,
    ],
    mesh=plsc.VectorSubcoreMesh(core_axis_name="c", subcore_axis_name="s"),
)
def _gather_rows(codes_hbm, table_hbm, out_hbm, idx_v, rows_v, sem):
    wid = lax.axis_index("s") * _NC + lax.axis_index("c")
    base = wid * _BPW
    for c in range(_BPW // _GCH):
        off = base + c * _GCH
        pltpu.sync_copy(codes_hbm.at[pl.ds(off, _GCH)], idx_v)
        pltpu.async_copy(table_hbm.at[idx_v], rows_v, sem).wait()
        pltpu.sync_copy(rows_v, out_hbm.at[pl.ds(off, _GCH)])


def _out_loss_body(x_ref, q_ref, out_ref, loss_ref, acc_s):
    i = pl.program_id(0)
    xv = x_ref[...]
    qv = q_ref[...]
    out_ref[...] = xv + (qv - xv)
    diff = xv - qv
    part = jnp.sum(diff * diff)

    @pl.when(i == 0)
    def _():
        acc_s[0, 0] = part

    @pl.when(i > 0)
    def _():
        acc_s[0, 0] = acc_s[0, 0] + part

    @pl.when(i == pl.num_programs(0) - 1)
    def _():
        total = acc_s[0, 0]
        mean = total / jnp.float32(_N * _D)
        loss_ref[...] = jnp.reshape(mean + jnp.float32(_BETA) * mean, (1, 1))


def _out_and_loss(flat_x, q):
    out, loss = pl.pallas_call(
        _out_loss_body,
        grid=(_N // _NT,),
        in_specs=[
            pl.BlockSpec((_NT, _D), lambda i: (i, 0)),
            pl.BlockSpec((_NT, _D), lambda i: (i, 0)),
        ],
        out_specs=[
            pl.BlockSpec((_NT, _D), lambda i: (i, 0)),
            pl.BlockSpec((1, 1), lambda i: (0, 0)),
        ],
        out_shape=[
            jax.ShapeDtypeStruct((_N, _D), jnp.float32),
            jax.ShapeDtypeStruct((1, 1), jnp.float32),
        ],
        scratch_shapes=[pltpu.SMEM((1, 1), jnp.float32)],
    )(flat_x, q)
    return out, loss[0, 0]


def kernel(x, embeddings):
    flat = jnp.reshape(x, (-1, _D))
    x2 = jnp.sum(flat ** 2, axis=1, keepdims=True)
    e2 = jnp.sum(embeddings ** 2, axis=0, keepdims=True)
    emb_bf = embeddings.astype(jnp.bfloat16)
    flatm2T_bf = (-2.0 * flat.T).astype(jnp.bfloat16)
    embT_bf = emb_bf.T
    x2r = jnp.reshape(x2, (1, _N))
    e2c = jnp.reshape(e2, (_K, 1))
    codes = _compute_codes(flatm2T_bf, embT_bf, x2r, e2c)
    table = embT_bf.astype(jnp.float32)
    q = _gather_rows(codes, table)
    out_flat, loss = _out_and_loss(flat, q)
    return jnp.reshape(out_flat, x.shape), loss


# speedup vs baseline: 2.4752x; 1.0407x over previous
import functools

import jax
import jax.numpy as jnp
from jax import lax
from jax.experimental import pallas as pl
from jax.experimental.pallas import tpu as pltpu
from jax.experimental.pallas import tpu_sc as plsc

_D = 256
_K = 8192
_N = 16384
_BETA = 0.25

_NT = 512
_KT = 1024
_NKS = _K // _KT
_B1_STEP, _B1_ROW = 2, 768
_B2_STEP, _B2_ROW = 5, 512
_BIG = 3.0e38


def _bf16_round(v):
    return v.astype(jnp.bfloat16).astype(jnp.float32)


def _colmin(d, base):
    lmin = jnp.min(d, axis=0, keepdims=True)
    iota = lax.broadcasted_iota(jnp.int32, (d.shape[0], 1), 0).astype(jnp.float32)
    lidx = jnp.min(jnp.where(d == lmin, iota, _BIG), axis=0, keepdims=True)
    return lmin, lidx + jnp.float32(base)


def _argmin_body(x2_ref, flatT_ref, embT_ref, e2_ref, codes_ref,
                 min_s, idx_s, gmin_s, gidx_s):
    k = pl.program_id(1)
    mm = jax.lax.dot_general(
        embT_ref[...], flatT_ref[...], (((1,), (1,)), ((), ())),
        preferred_element_type=jnp.float32)
    d = (x2_ref[...] + mm) + e2_ref[...]
    base = k * _KT

    def merge_chunk(lmin, lidx):
        upd = lmin < min_s[...]
        min_s[...] = jnp.where(upd, lmin, min_s[...])
        idx_s[...] = jnp.where(upd, lidx, idx_s[...])

    def fold_global(first):
        if first:
            gmin_s[...] = _bf16_round(min_s[...])
            gidx_s[...] = idx_s[...]
        else:
            upd = min_s[...] < gmin_s[...]
            gmin_s[...] = jnp.where(upd, _bf16_round(min_s[...]), gmin_s[...])
            gidx_s[...] = jnp.where(upd, idx_s[...], gidx_s[...])

    @pl.when(k == 0)
    def _():
        lmin, lidx = _colmin(d, base)
        min_s[...] = lmin
        idx_s[...] = lidx

    @pl.when((k == 1) | (k == 3) | (k == 4) | (k == 6) | (k == _NKS - 1))
    def _():
        merge_chunk(*_colmin(d, base))

    @pl.when(k == _B1_STEP)
    def _():
        merge_chunk(*_colmin(d[:_B1_ROW], base))
        fold_global(True)
        lmin, lidx = _colmin(d[_B1_ROW:], base + _B1_ROW)
        min_s[...] = lmin
        idx_s[...] = lidx

    @pl.when(k == _B2_STEP)
    def _():
        merge_chunk(*_colmin(d[:_B2_ROW], base))
        fold_global(False)
        lmin, lidx = _colmin(d[_B2_ROW:], base + _B2_ROW)
        min_s[...] = lmin
        idx_s[...] = lidx

    @pl.when(k == _NKS - 1)
    def _():
        fold_global(False)
        codes_ref[...] = gidx_s[...].astype(jnp.int32)


def _compute_codes(flatm2T_bf, embT_bf, x2r, e2c):
    grid = (_N // _NT, _NKS)
    codes = pl.pallas_call(
        _argmin_body,
        grid=grid,
        in_specs=[
            pl.BlockSpec((1, _NT), lambda i, k: (0, i)),
            pl.BlockSpec((_NT, _D), lambda i, k: (i, 0)),
            pl.BlockSpec((_KT, _D), lambda i, k: (k, 0)),
            pl.BlockSpec((_KT, 1), lambda i, k: (k, 0)),
        ],
        out_specs=pl.BlockSpec((1, _NT), lambda i, k: (0, i)),
        out_shape=jax.ShapeDtypeStruct((1, _N), jnp.int32),
        scratch_shapes=[
            pltpu.VMEM((1, _NT), jnp.float32),
            pltpu.VMEM((1, _NT), jnp.float32),
            pltpu.VMEM((1, _NT), jnp.float32),
            pltpu.VMEM((1, _NT), jnp.float32),
        ],
    )(x2r, flatm2T_bf, embT_bf, e2c)
    return jnp.reshape(codes, (_N,))


_SC_INFO = plsc.get_sparse_core_info()
_NC = _SC_INFO.num_cores
_NS = _SC_INFO.num_subcores
_NW = _NC * _NS
_BPW = _N // _NW
_GCH = 128


@functools.partial(
    pl.kernel,
    out_type=jax.ShapeDtypeStruct((_N, _D), jnp.float32),
    scratch_types=[
        pltpu.VMEM((_GCH,), jnp.int32),
        pltpu.VMEM((_GCH, _D), jnp.float32),
        pltpu.SemaphoreType.DMA,
    ],
    mesh=plsc.VectorSubcoreMesh(core_axis_name="c", subcore_axis_name="s"),
)
def _gather_rows(codes_hbm, table_hbm, out_hbm, idx_v, rows_v, sem):
    wid = lax.axis_index("s") * _NC + lax.axis_index("c")
    base = wid * _BPW
    for c in range(_BPW // _GCH):
        off = base + c * _GCH
        pltpu.sync_copy(codes_hbm.at[pl.ds(off, _GCH)], idx_v)
        pltpu.async_copy(table_hbm.at[idx_v], rows_v, sem).wait()
        pltpu.sync_copy(rows_v, out_hbm.at[pl.ds(off, _GCH)])


def _out_loss_body(x_ref, q_ref, out_ref, loss_ref, acc_s):
    i = pl.program_id(0)
    xv = x_ref[...]
    qv = q_ref[...]
    out_ref[...] = xv + (qv - xv)
    diff = xv - qv
    part = jnp.sum(diff * diff)

    @pl.when(i == 0)
    def _():
        acc_s[0, 0] = part

    @pl.when(i > 0)
    def _():
        acc_s[0, 0] = acc_s[0, 0] + part

    @pl.when(i == pl.num_programs(0) - 1)
    def _():
        total = acc_s[0, 0]
        mean = total / jnp.float32(_N * _D)
        loss_ref[...] = jnp.reshape(mean + jnp.float32(_BETA) * mean, (1, 1))


def _out_and_loss(flat_x, q):
    out, loss = pl.pallas_call(
        _out_loss_body,
        grid=(_N // _NT,),
        in_specs=[
            pl.BlockSpec((_NT, _D), lambda i: (i, 0)),
            pl.BlockSpec((_NT, _D), lambda i: (i, 0)),
        ],
        out_specs=[
            pl.BlockSpec((_NT, _D), lambda i: (i, 0)),
            pl.BlockSpec((1, 1), lambda i: (0, 0)),
        ],
        out_shape=[
            jax.ShapeDtypeStruct((_N, _D), jnp.float32),
            jax.ShapeDtypeStruct((1, 1), jnp.float32),
        ],
        scratch_shapes=[pltpu.SMEM((1, 1), jnp.float32)],
    )(flat_x, q)
    return out, loss[0, 0]


def kernel(x, embeddings):
    flat = jnp.reshape(x, (-1, _D))
    x2 = jnp.sum(flat ** 2, axis=1, keepdims=True)
    e2 = jnp.sum(embeddings ** 2, axis=0, keepdims=True)
    emb_bf = embeddings.astype(jnp.bfloat16)
    flatm2_bf = (-2.0 * flat).astype(jnp.bfloat16)
    embT_bf = emb_bf.T
    x2r = jnp.reshape(x2, (1, _N))
    e2c = jnp.reshape(e2, (_K, 1))
    codes = _compute_codes(flatm2_bf, embT_bf, x2r, e2c)
    table = embT_bf.astype(jnp.float32)
    q = _gather_rows(codes, table)
    out_flat, loss = _out_and_loss(flat, q)
    return jnp.reshape(out_flat, x.shape), loss


# speedup vs baseline: 3.0461x; 1.2306x over previous
import functools

import jax
import jax.numpy as jnp
from jax import lax
from jax.experimental import pallas as pl
from jax.experimental.pallas import tpu as pltpu
from jax.experimental.pallas import tpu_sc as plsc

_D = 256
_K = 8192
_N = 16384
_BETA = 0.25

_NT = 512
_KT = 2048
_NKS = _K // _KT
_B1_STEP, _B1_ROW = 1, 768
_B2_STEP, _B2_ROW = 2, 1536
_BIG = 3.0e38


def _bf16_round(v):
    return v.astype(jnp.bfloat16).astype(jnp.float32)


def _colmin(d, base):
    lmin = jnp.min(d, axis=0, keepdims=True)
    iota = lax.broadcasted_iota(jnp.int32, (d.shape[0], 1), 0).astype(jnp.float32)
    lidx = jnp.min(jnp.where(d == lmin, iota, _BIG), axis=0, keepdims=True)
    return lmin, lidx + jnp.float32(base)


def _argmin_body(x2_ref, flatT_ref, embT_ref, e2_ref, codes_ref,
                 min_s, idx_s, gmin_s, gidx_s):
    k = pl.program_id(1)
    mm = jax.lax.dot_general(
        embT_ref[...], flatT_ref[...], (((1,), (1,)), ((), ())),
        preferred_element_type=jnp.float32)
    d = (x2_ref[...] + mm) + e2_ref[...]
    base = k * _KT

    def merge_chunk(lmin, lidx):
        upd = lmin < min_s[...]
        min_s[...] = jnp.where(upd, lmin, min_s[...])
        idx_s[...] = jnp.where(upd, lidx, idx_s[...])

    def fold_global(first):
        if first:
            gmin_s[...] = _bf16_round(min_s[...])
            gidx_s[...] = idx_s[...]
        else:
            upd = min_s[...] < gmin_s[...]
            gmin_s[...] = jnp.where(upd, _bf16_round(min_s[...]), gmin_s[...])
            gidx_s[...] = jnp.where(upd, idx_s[...], gidx_s[...])

    @pl.when(k == 0)
    def _():
        lmin, lidx = _colmin(d, base)
        min_s[...] = lmin
        idx_s[...] = lidx

    @pl.when((k != 0) & (k != _B1_STEP) & (k != _B2_STEP))
    def _():
        merge_chunk(*_colmin(d, base))

    @pl.when(k == _B1_STEP)
    def _():
        merge_chunk(*_colmin(d[:_B1_ROW], base))
        fold_global(True)
        lmin, lidx = _colmin(d[_B1_ROW:], base + _B1_ROW)
        min_s[...] = lmin
        idx_s[...] = lidx

    @pl.when(k == _B2_STEP)
    def _():
        merge_chunk(*_colmin(d[:_B2_ROW], base))
        fold_global(False)
        lmin, lidx = _colmin(d[_B2_ROW:], base + _B2_ROW)
        min_s[...] = lmin
        idx_s[...] = lidx

    @pl.when(k == _NKS - 1)
    def _():
        fold_global(False)
        codes_ref[...] = gidx_s[...].astype(jnp.int32)


def _compute_codes(flatm2T_bf, embT_bf, x2r, e2c):
    grid = (_N // _NT, _NKS)
    codes = pl.pallas_call(
        _argmin_body,
        grid=grid,
        in_specs=[
            pl.BlockSpec((1, _NT), lambda i, k: (0, i)),
            pl.BlockSpec((_NT, _D), lambda i, k: (i, 0)),
            pl.BlockSpec((_KT, _D), lambda i, k: (k, 0)),
            pl.BlockSpec((_KT, 1), lambda i, k: (k, 0)),
        ],
        out_specs=pl.BlockSpec((1, _NT), lambda i, k: (0, i)),
        out_shape=jax.ShapeDtypeStruct((1, _N), jnp.int32),
        scratch_shapes=[
            pltpu.VMEM((1, _NT), jnp.float32),
            pltpu.VMEM((1, _NT), jnp.float32),
            pltpu.VMEM((1, _NT), jnp.float32),
            pltpu.VMEM((1, _NT), jnp.float32),
        ],
    )(x2r, flatm2T_bf, embT_bf, e2c)
    return jnp.reshape(codes, (_N,))


_SC_INFO = plsc.get_sparse_core_info()
_NC = _SC_INFO.num_cores
_NS = _SC_INFO.num_subcores
_NW = _NC * _NS
_BPW = _N // _NW
_GCH = 128


@functools.partial(
    pl.kernel,
    out_type=jax.ShapeDtypeStruct((_N, _D), jnp.float32),
    scratch_types=[
        pltpu.VMEM((_GCH,), jnp.int32),
        pltpu.VMEM((_GCH, _D), jnp.float32),
        pltpu.SemaphoreType.DMA,
    ],
    mesh=plsc.VectorSubcoreMesh(core_axis_name="c", subcore_axis_name="s"),
)
def _gather_rows(codes_hbm, table_hbm, out_hbm, idx_v, rows_v, sem):
    wid = lax.axis_index("s") * _NC + lax.axis_index("c")
    base = wid * _BPW
    for c in range(_BPW // _GCH):
        off = base + c * _GCH
        pltpu.sync_copy(codes_hbm.at[pl.ds(off, _GCH)], idx_v)
        pltpu.async_copy(table_hbm.at[idx_v], rows_v, sem).wait()
        pltpu.sync_copy(rows_v, out_hbm.at[pl.ds(off, _GCH)])


def _out_loss_body(x_ref, q_ref, out_ref, loss_ref, acc_s):
    i = pl.program_id(0)
    xv = x_ref[...]
    qv = q_ref[...]
    out_ref[...] = xv + (qv - xv)
    diff = xv - qv
    part = jnp.sum(diff * diff)

    @pl.when(i == 0)
    def _():
        acc_s[0, 0] = part

    @pl.when(i > 0)
    def _():
        acc_s[0, 0] = acc_s[0, 0] + part

    @pl.when(i == pl.num_programs(0) - 1)
    def _():
        total = acc_s[0, 0]
        mean = total / jnp.float32(_N * _D)
        loss_ref[...] = jnp.reshape(mean + jnp.float32(_BETA) * mean, (1, 1))


def _out_and_loss(flat_x, q):
    out, loss = pl.pallas_call(
        _out_loss_body,
        grid=(_N // _NT,),
        in_specs=[
            pl.BlockSpec((_NT, _D), lambda i: (i, 0)),
            pl.BlockSpec((_NT, _D), lambda i: (i, 0)),
        ],
        out_specs=[
            pl.BlockSpec((_NT, _D), lambda i: (i, 0)),
            pl.BlockSpec((1, 1), lambda i: (0, 0)),
        ],
        out_shape=[
            jax.ShapeDtypeStruct((_N, _D), jnp.float32),
            jax.ShapeDtypeStruct((1, 1), jnp.float32),
        ],
        scratch_shapes=[pltpu.SMEM((1, 1), jnp.float32)],
    )(flat_x, q)
    return out, loss[0, 0]


def kernel(x, embeddings):
    flat = jnp.reshape(x, (-1, _D))
    x2 = jnp.sum(flat ** 2, axis=1, keepdims=True)
    e2 = jnp.sum(embeddings ** 2, axis=0, keepdims=True)
    emb_bf = embeddings.astype(jnp.bfloat16)
    flatm2_bf = (-2.0 * flat).astype(jnp.bfloat16)
    embT_bf = emb_bf.T
    x2r = jnp.reshape(x2, (1, _N))
    e2c = jnp.reshape(e2, (_K, 1))
    codes = _compute_codes(flatm2_bf, embT_bf, x2r, e2c)
    table = embT_bf.astype(jnp.float32)
    q = _gather_rows(codes, table)
    out_flat, loss = _out_and_loss(flat, q)
    return jnp.reshape(out_flat, x.shape), loss
